# baseline (speedup 1.0000x reference)
import functools

import jax
import jax.numpy as jnp
from jax import lax
from jax.experimental import pallas as pl
from jax.experimental.pallas import tpu as pltpu
from jax.experimental.pallas import tpu_sc as plsc

N = 10000
E = 160000
D_WORD = 300
D_RGB = 512
MID = 256
ATTR = 64
OUT = 256
EMBED = 512

NC = 2
NS = 16
L = 16

Q = MID // 4
QV = Q // L
ET = E // NS
CHUNK = 80
NCHUNK = ET // CHUNK
NBUF = 4
NPAD = 10240
ROWS_PER_TILE = NPAD // NS
WB = 32

_MAGIC = 6554

_mesh = plsc.VectorSubcoreMesh(core_axis_name="c", subcore_axis_name="s",
                               num_cores=NC, num_subcores=NS)

_SC_PARAMS = pltpu.CompilerParams(needs_layout_passes=False,
                                  use_tc_tiling_on_sc=False)

_SPLAT_DN = lax.GatherDimensionNumbers(
    offset_dims=(), collapsed_slice_dims=(0,), start_index_map=(0,))


def _splat(vec, l):
  return lax.gather(vec, jnp.full((L, 1), l, jnp.int32), _SPLAT_DN, (1,),
                    mode=lax.GatherScatterMode.PROMISE_IN_BOUNDS)


def _row_col(n16):
  r = lax.shift_right_logical(n16 * _MAGIC, 22)
  return r, n16 - r * ROWS_PER_TILE


def _rsqrt_bits(x):
  i = lax.bitcast_convert_type(x, jnp.int32)
  y = lax.bitcast_convert_type(
      jnp.int32(0x5F3759DF) - lax.shift_right_logical(i, 1), jnp.float32)
  for _ in range(3):
    y = y * (1.5 - 0.5 * x * y * y)
  return y


def _quarters_phase(c, s, g_all, out_all, src_v, dst_v, scale_v, rows, srows,
                    wbuf, zbuf, acc, gsems, ssems):
  zero = jnp.zeros((L,), jnp.float32)

  def zbuf_body(i, _):
    zbuf[i // QV, pl.ds((i % QV) * L, L)] = zero
    return _

  lax.fori_loop(0, WB * QV, zbuf_body, None)

  def multiply(j, b):
    def group_body(gi, _g):
      ew16 = scale_v[j, pl.ds(gi * L, L)]
      for l in range(L):
        sv = _splat(ew16, l)
        e = gi * L + l
        for p in range(QV):
          srows[b][e, pl.ds(p * L, L)] = rows[b][e, pl.ds(p * L, L)] * sv
      return _g

    lax.fori_loop(0, CHUNK // L, group_body, None)

  def run_quarter(qq):
    g = g_all.at[qq, c]
    out = out_all.at[qq, c]

    def zacc_body(k, _):
      pltpu.sync_copy(zbuf, acc.at[pl.ds(s * ROWS_PER_TILE + k * WB, WB)])
      return _

    lax.fori_loop(0, ROWS_PER_TILE // WB, zacc_body, None)
    plsc.subcore_barrier()

    def start_gather(j, b):
      pltpu.async_copy(g.at[src_v.at[j]], rows[b], gsems[b])

    def wait_gather(j, b):
      pltpu.make_async_copy(g.at[src_v.at[j]], rows[b], gsems[b]).wait()

    def start_scatter(j, b):
      pltpu.async_copy(srows[b], acc.at[dst_v.at[j]], ssems[b], add=True)

    def wait_scatter(b):
      pltpu.make_async_copy(srows[b], acc.at[dst_v.at[0]], ssems[b]).wait()

    start_gather(0, 0)
    start_gather(1, 1)
    start_gather(2, 2)

    def quad_body(jj, _):
      for b in range(NBUF):
        j = jj * NBUF + b

        @pl.when(j < NCHUNK)
        def _():
          wait_gather(j, b)

          @pl.when(j >= NBUF)
          def _():
            wait_scatter(b)

          multiply(j, b)
          start_scatter(j, b)

          @pl.when(j + 3 < NCHUNK)
          def _():
            start_gather(j + 3, (b + 3) % NBUF)

      return _

    lax.fori_loop(0, (NCHUNK + NBUF - 1) // NBUF, quad_body, None)
    for jd in range(NCHUNK - NBUF, NCHUNK):
      wait_scatter(jd % NBUF)
    plsc.subcore_barrier()

    def wb_body(k, _):
      sl = pl.ds(s * ROWS_PER_TILE + k * WB, WB)
      pltpu.sync_copy(acc.at[sl], wbuf)
      pltpu.sync_copy(wbuf, out.at[sl])
      return _

    lax.fori_loop(0, ROWS_PER_TILE // WB, wb_body, None)
    plsc.subcore_barrier()

  run_quarter(0)
  run_quarter(1)


_EDGE_SCRATCH = [
    pltpu.VMEM((NCHUNK, CHUNK), jnp.int32),
    pltpu.VMEM((NCHUNK, CHUNK), jnp.int32),
    pltpu.VMEM((NCHUNK, CHUNK), jnp.float32),
    [pltpu.VMEM((CHUNK, Q), jnp.float32) for _ in range(NBUF)],
    [pltpu.VMEM((CHUNK, Q), jnp.float32) for _ in range(NBUF)],
    pltpu.VMEM((WB, Q), jnp.float32),
    pltpu.VMEM((WB, Q), jnp.float32),
    pltpu.VMEM_SHARED((NPAD, Q), jnp.float32),
    [pltpu.SemaphoreType.DMA for _ in range(NBUF)],
    [pltpu.SemaphoreType.DMA for _ in range(NBUF)],
]


@functools.partial(
    pl.kernel,
    out_type=[
        jax.ShapeDtypeStruct((2, NC, NPAD, Q), jnp.float32),
        jax.ShapeDtypeStruct((NS, ROWS_PER_TILE), jnp.float32),
        jax.ShapeDtypeStruct((NS, NCHUNK, CHUNK), jnp.float32),
    ],
    mesh=_mesh,
    scratch_types=_EDGE_SCRATCH + [
        pltpu.VMEM((NS, ROWS_PER_TILE), jnp.float32),
        pltpu.VMEM((ROWS_PER_TILE,), jnp.float32),
        pltpu.VMEM((1, L), jnp.int32),
        pltpu.VMEM_SHARED((NS, ROWS_PER_TILE), jnp.float32),
    ],
    compiler_params=_SC_PARAMS,
)
def _edge_kernel1(g_all, src_hbm, dst_hbm, ew_hbm, out_all, dinv_out, w2_out,
                  src_v, dst_v, ew_v, rows, srows, wbuf, zbuf, acc, gsems,
                  ssems, dp_v, dv, idx2, nd_sh):
  c = lax.axis_index("c")
  s = lax.axis_index("s")

  pltpu.sync_copy(src_hbm.at[s], src_v)
  pltpu.sync_copy(dst_hbm.at[s], dst_v)
  pltpu.sync_copy(ew_hbm.at[s], ew_v)

  zero = jnp.zeros((L,), jnp.float32)
  idx2[0, pl.ds(0, L)] = lax.iota(jnp.int32, L)

  def zrow_body(i, _):
    dv[pl.ds(i * L, L)] = zero
    return _

  lax.fori_loop(0, ROWS_PER_TILE // L, zrow_body, None)

  def dp_zero(i, _):
    dp_v[i // (ROWS_PER_TILE // L), pl.ds((i % (ROWS_PER_TILE // L)) * L, L)] = zero
    return _

  lax.fori_loop(0, NPAD // L, dp_zero, None)

  def deg_body(j, _):
    for k in range(CHUNK // L):
      d16 = dst_v[j, pl.ds(k * L, L)]
      e16 = ew_v[j, pl.ds(k * L, L)]
      r16, c16 = _row_col(d16)
      plsc.addupdate_scatter(dp_v, [r16, c16], e16)
    return _

  lax.fori_loop(0, NCHUNK, deg_body, None)
  pltpu.sync_copy(dv, nd_sh.at[s])
  plsc.subcore_barrier()
  pltpu.sync_copy(dp_v, nd_sh.at[idx2.at[0]], add=True)
  plsc.subcore_barrier()
  pltpu.sync_copy(nd_sh.at[s], dv)

  def dinv_body(i, _):
    x = dv[pl.ds(i * L, L)] + 1.0
    dv[pl.ds(i * L, L)] = _rsqrt_bits(x)
    return _

  lax.fori_loop(0, ROWS_PER_TILE // L, dinv_body, None)
  pltpu.sync_copy(dv, nd_sh.at[s])

  @pl.when(c == 0)
  def _():
    pltpu.sync_copy(dv, dinv_out.at[s])

  plsc.subcore_barrier()
  pltpu.sync_copy(nd_sh, dp_v)

  def w2_body(j, _):
    for k in range(CHUNK // L):
      s16 = src_v[j, pl.ds(k * L, L)]
      r16, c16 = _row_col(s16)
      d16 = plsc.load_gather(dp_v, [r16, c16])
      ew_v[j, pl.ds(k * L, L)] = ew_v[j, pl.ds(k * L, L)] * d16
    return _

  lax.fori_loop(0, NCHUNK, w2_body, None)

  @pl.when(c == 0)
  def _():
    pltpu.sync_copy(ew_v, w2_out.at[s])

  _quarters_phase(c, s, g_all, out_all, src_v, dst_v, ew_v, rows, srows,
                  wbuf, zbuf, acc, gsems, ssems)


@functools.partial(
    pl.kernel,
    out_type=jax.ShapeDtypeStruct((2, NC, NPAD, Q), jnp.float32),
    mesh=_mesh,
    scratch_types=_EDGE_SCRATCH,
    compiler_params=_SC_PARAMS,
)
def _edge_kernel2(g_all, src_hbm, dst_hbm, w2_hbm, out_all,
                  src_v, dst_v, w2_v, rows, srows, wbuf, zbuf, acc, gsems,
                  ssems):
  c = lax.axis_index("c")
  s = lax.axis_index("s")

  pltpu.sync_copy(src_hbm.at[s], src_v)
  pltpu.sync_copy(dst_hbm.at[s], dst_v)
  pltpu.sync_copy(w2_hbm.at[s], w2_v)

  _quarters_phase(c, s, g_all, out_all, src_v, dst_v, w2_v, rows, srows,
                  wbuf, zbuf, acc, gsems, ssems)


RB = 400
NRB = N // RB


def _write_quarters(ref, mat):
  for qq in range(2):
    for cc in range(NC):
      ref[qq, cc] = mat[:, (qq * NC + cc) * Q:(qq * NC + cc + 1) * Q]


def _read_quarters(ref):
  return jnp.concatenate(
      [ref[qq, cc] for qq in range(2) for cc in range(NC)], axis=1)


_STACK_SPEC = pl.BlockSpec((2, NC, RB, Q), lambda i: (0, 0, i, 0))


def _tc_pre_body(x_ref, ww_ref, bw_ref, wr_ref, br_ref, w1_ref, gq_ref):
  xb = x_ref[...]
  word = jnp.dot(xb[:, :D_WORD], ww_ref[...],
                 preferred_element_type=jnp.float32) + bw_ref[...]
  rgb = jnp.dot(xb[:, D_WORD:], wr_ref[...],
                preferred_element_type=jnp.float32) + br_ref[...]
  h = jnp.maximum(jnp.concatenate([word, rgb], axis=1), 0.0)
  g = jnp.dot(h, w1_ref[...], preferred_element_type=jnp.float32)
  _write_quarters(gq_ref, g)


_tc_pre = pl.pallas_call(
    _tc_pre_body,
    grid=(NRB,),
    in_specs=[
        pl.BlockSpec((RB, D_WORD + D_RGB), lambda i: (i, 0)),
        pl.BlockSpec((D_WORD, MID // 2), lambda i: (0, 0)),
        pl.BlockSpec((1, MID // 2), lambda i: (0, 0)),
        pl.BlockSpec((D_RGB, MID // 2), lambda i: (0, 0)),
        pl.BlockSpec((1, MID // 2), lambda i: (0, 0)),
        pl.BlockSpec((MID, MID), lambda i: (0, 0)),
    ],
    out_specs=_STACK_SPEC,
    out_shape=jax.ShapeDtypeStruct((2, NC, N, Q), jnp.float32),
)


def _tc_mid_body(acc_ref, gq_ref, dinv_ref, b1_ref, attr_ref, w2_ref,
                 oq_ref):
  dinv = dinv_ref[...]
  acc = _read_quarters(acc_ref)
  gp = _read_quarters(gq_ref)
  h2 = jnp.maximum(dinv * (acc + dinv * gp) + b1_ref[...], 0.0)
  cat = jnp.concatenate([h2, attr_ref[...]], axis=1)
  g2 = jnp.dot(cat, w2_ref[...], preferred_element_type=jnp.float32)
  _write_quarters(oq_ref, g2)


_tc_mid = pl.pallas_call(
    _tc_mid_body,
    grid=(NRB,),
    in_specs=[
        _STACK_SPEC,
        _STACK_SPEC,
        pl.BlockSpec((RB, 1), lambda i: (i, 0)),
        pl.BlockSpec((1, MID), lambda i: (0, 0)),
        pl.BlockSpec((RB, ATTR), lambda i: (i, 0)),
        pl.BlockSpec((MID + ATTR, OUT), lambda i: (0, 0)),
    ],
    out_specs=_STACK_SPEC,
    out_shape=jax.ShapeDtypeStruct((2, NC, N, Q), jnp.float32),
)


def _tc_post_body(acc_ref, gq_ref, dinv_ref, b2_ref, attr_ref, wf_ref,
                  bf_ref, out_ref, psum_ref):
  i = pl.program_id(0)
  dinv = dinv_ref[...]
  acc = _read_quarters(acc_ref)
  gp = _read_quarters(gq_ref)
  o = jnp.maximum(dinv * (acc + dinv * gp) + b2_ref[...], 0.0)
  cat = jnp.concatenate([o, attr_ref[...]], axis=1)
  blk_sum = jnp.sum(cat, axis=0, keepdims=True)

  @pl.when(i == 0)
  def _():
    psum_ref[...] = jnp.zeros_like(psum_ref)

  psum_ref[...] += blk_sum

  @pl.when(i == NRB - 1)
  def _():
    pooled = psum_ref[...] * (1.0 / N)
    out_ref[...] = jnp.maximum(
        jnp.dot(pooled, wf_ref[...], preferred_element_type=jnp.float32)
        + bf_ref[...], 0.0)


_tc_post = pl.pallas_call(
    _tc_post_body,
    grid=(NRB,),
    in_specs=[
        _STACK_SPEC,
        _STACK_SPEC,
        pl.BlockSpec((RB, 1), lambda i: (i, 0)),
        pl.BlockSpec((1, OUT), lambda i: (0, 0)),
        pl.BlockSpec((RB, ATTR), lambda i: (i, 0)),
        pl.BlockSpec((OUT + ATTR, EMBED), lambda i: (0, 0)),
        pl.BlockSpec((1, EMBED), lambda i: (0, 0)),
    ],
    out_specs=pl.BlockSpec((1, EMBED), lambda i: (0, 0)),
    out_shape=jax.ShapeDtypeStruct((1, EMBED), jnp.float32),
    scratch_shapes=[pltpu.VMEM((1, OUT + ATTR), jnp.float32)],
)


@jax.jit
def kernel(x, attributes, edge_index, edge_weight, W_word, b_word, W_rgb,
           b_rgb, W1, b1, W2, b2, Wf, bf):
  src = edge_index[0].astype(jnp.int32)
  dst = edge_index[1].astype(jnp.int32)
  ew = edge_weight.astype(jnp.float32)

  src_r = src.reshape(NS, NCHUNK, CHUNK)
  dst_r = dst.reshape(NS, NCHUNK, CHUNK)
  ew_r = ew.reshape(NS, NCHUNK, CHUNK)

  bw = b_word.reshape(1, MID // 2)
  br = b_rgb.reshape(1, MID // 2)
  b1r = b1.reshape(1, MID)
  b2r = b2.reshape(1, OUT)
  bfr = bf.reshape(1, EMBED)

  g1q = _tc_pre(x, W_word, bw, W_rgb, br, W1)
  a1, dinv_t, w2_r = _edge_kernel1(g1q, src_r, dst_r, ew_r)
  dinv_col = dinv_t.reshape(NPAD)[:N].reshape(N, 1)
  g2q = _tc_mid(a1, g1q, dinv_col, b1r, attributes, W2)
  a2 = _edge_kernel2(g2q, src_r, dst_r, w2_r)
  return _tc_post(a2, g2q, dinv_col, b2r, attributes, Wf, bfr)
